# Initial kernel scaffold; baseline (speedup 1.0000x reference)
import jax
import jax.numpy as jnp
from jax.experimental import pallas as pl


def kernel(x_inp_core, edge_index_core, edge_attr_core, x_upd_core, Z_core, Z_block, node2graph_core, W_embed, b_embed, W_edge, b_edge, W_msg, b_msg, W1, b1, W2, b2):
    raise NotImplementedError("write your pallas kernel here")


# speedup vs baseline: 3.2674x; 3.2674x over previous
import jax
import jax.numpy as jnp
from jax import lax
from jax.experimental import pallas as pl
from jax.experimental.pallas import tpu as pltpu
from jax.experimental.pallas import tpu_sc as plsc

V = 10000
E = 320000
G = 128
D = 128
DE = 16
NL = 4

NC = 2
NS = 16
NW = NC * NS
EPW = E // NW
C = 80
NCHUNK = EPW // C
SR = 624
TAIL = V - NS * SR

_f32 = jnp.float32
_bf16 = jnp.bfloat16


def _bdot(a, b):
    return lax.dot(a.astype(_bf16), b.astype(_bf16),
                   preferred_element_type=_f32)


def _xdot(a, b):
    return lax.dot(a, b, preferred_element_type=_f32,
                   precision=jax.lax.Precision.HIGHEST)


def _embed_body(xu_ref, zc_ref, wx_ref, wz_ref, b_ref, n2g_ref, out_ref):
    gids = lax.broadcasted_iota(jnp.int32, (1, G), 1)
    oh = (n2g_ref[...] == gids).astype(_f32)
    g = _xdot(oh, zc_ref[...])
    acc = _bdot(xu_ref[...], wx_ref[...]) + _bdot(g, wz_ref[...]) + b_ref[...]
    out_ref[...] = jnp.maximum(acc, 0.0)


def _edgefeat_body(attr_ref, w_ref, b_ref, out_ref):
    out_ref[...] = jnp.maximum(
        _bdot(attr_ref[...], w_ref[...]) + b_ref[...], 0.0)


def _update_body(h_ref, agg_ref, wt_ref, wb_ref, b_ref, out_ref):
    a = agg_ref[0] + agg_ref[1]
    acc = _bdot(h_ref[...], wt_ref[...]) + _bdot(a, wb_ref[...]) + b_ref[...]
    out_ref[...] = h_ref[...] + jnp.maximum(acc, 0.0)


def _head_body(h_ref, xi_ref, w1h_ref, w1x_ref, b1_ref, w2_ref, b2_ref,
               n2g_ref, out_ref):
    hid = jnp.maximum(
        _bdot(h_ref[...], w1h_ref[...]) + _bdot(xi_ref[...], w1x_ref[...])
        + b1_ref[...], 0.0)
    logit = _bdot(hid, w2_ref[...]) + b2_ref[...]
    gids = lax.broadcasted_iota(jnp.int32, (1, G), 1)
    oh = (n2g_ref[...] == gids).astype(_f32)
    masked = jnp.where(oh > 0.0, logit, -1e30)
    mx = jnp.max(masked, axis=0, keepdims=True)
    mxv = jnp.sum(oh * mx, axis=1, keepdims=True)
    ex = jnp.exp(logit - mxv)
    den = jnp.sum(oh * ex, axis=0, keepdims=True)
    denv = jnp.sum(oh * den, axis=1, keepdims=True)
    out_ref[...] = ex / denv


def _sc_edge_body(h_hbm, e_hbm, src_hbm, dst_hbm, out_hbm,
                  sidx, didx, ebuf, hbuf, agg_sh, sem_e, sem_h):
    cid = lax.axis_index("c")
    sid = lax.axis_index("s")
    wid = sid * NC + cid

    def zrow(r, carry):
        for k in range(D // 16):
            ebuf[r, pl.ds(k * 16, 16)] = jnp.zeros((16,), _f32)
        return carry
    lax.fori_loop(0, C, zrow, 0)
    row0 = sid * SR
    nfull = SR // C
    for q in range(nfull):
        pltpu.sync_copy(ebuf, agg_sh.at[pl.ds(row0 + q * C, C)])
    rem = SR - nfull * C
    if rem:
        pltpu.sync_copy(ebuf.at[pl.ds(0, rem)],
                        agg_sh.at[pl.ds(row0 + nfull * C, rem)])

    @pl.when(sid == NS - 1)
    def _zero_tail():
        pltpu.sync_copy(ebuf.at[pl.ds(0, TAIL)],
                        agg_sh.at[pl.ds(NS * SR, TAIL)])
    plsc.subcore_barrier()

    def chunk(j, carry):
        base = wid * EPW + j * C
        pltpu.sync_copy(src_hbm.at[pl.ds(base, C)], sidx)
        pltpu.sync_copy(dst_hbm.at[pl.ds(base, C)], didx)
        cpe = pltpu.async_copy(e_hbm.at[pl.ds(base, C)], ebuf, sem_e)
        cph = pltpu.async_copy(h_hbm.at[sidx], hbuf, sem_h)
        cpe.wait()
        cph.wait()

        def row(r, c2):
            for k in range(D // 16):
                sl = pl.ds(k * 16, 16)
                hbuf[r, sl] = jnp.maximum(hbuf[r, sl] + ebuf[r, sl], 0.0)
            return c2
        lax.fori_loop(0, C, row, 0)
        pltpu.sync_copy(hbuf, agg_sh.at[didx], add=True)
        return carry
    lax.fori_loop(0, NCHUNK, chunk, 0)

    plsc.subcore_barrier()
    pltpu.sync_copy(agg_sh.at[pl.ds(row0, SR)],
                    out_hbm.at[cid, pl.ds(row0, SR)])

    @pl.when(sid == NS - 1)
    def _write_tail():
        pltpu.sync_copy(agg_sh.at[pl.ds(NS * SR, TAIL)],
                        out_hbm.at[cid, pl.ds(NS * SR, TAIL)])


def _make_sc_edge():
    return pl.kernel(
        _sc_edge_body,
        out_type=jax.ShapeDtypeStruct((NC, V, D), _f32),
        mesh=plsc.VectorSubcoreMesh(core_axis_name="c", subcore_axis_name="s",
                                    num_cores=NC, num_subcores=NS),
        scratch_types=[
            pltpu.VMEM((C,), jnp.int32),
            pltpu.VMEM((C,), jnp.int32),
            pltpu.VMEM((C, D), _f32),
            pltpu.VMEM((C, D), _f32),
            pltpu.VMEM_SHARED((V, D), _f32),
            pltpu.SemaphoreType.DMA,
            pltpu.SemaphoreType.DMA,
        ],
    )


def kernel(x_inp_core, edge_index_core, edge_attr_core, x_upd_core, Z_core,
           Z_block, node2graph_core, W_embed, b_embed, W_edge, b_edge, W_msg,
           b_msg, W1, b1, W2, b2):
    src = edge_index_core[0]
    dst = edge_index_core[1]
    Zcat = jnp.concatenate([Z_core, Z_block], axis=-1)
    n2g2 = node2graph_core.reshape(V, 1)

    h = pl.pallas_call(
        _embed_body,
        out_shape=jax.ShapeDtypeStruct((V, D), _f32),
    )(x_upd_core, Zcat, W_embed[:D], W_embed[D:], b_embed.reshape(1, D), n2g2)

    BE = 4000
    e = pl.pallas_call(
        _edgefeat_body,
        grid=(E // BE,),
        in_specs=[pl.BlockSpec((BE, DE), lambda i: (i, 0)),
                  pl.BlockSpec((DE, D), lambda i: (0, 0)),
                  pl.BlockSpec((1, D), lambda i: (0, 0))],
        out_specs=pl.BlockSpec((BE, D), lambda i: (i, 0)),
        out_shape=jax.ShapeDtypeStruct((E, D), _f32),
    )(edge_attr_core, W_edge, b_edge.reshape(1, D))

    sc_edge = _make_sc_edge()
    for l in range(NL):
        agg = sc_edge(h, e, src, dst)
        h = pl.pallas_call(
            _update_body,
            out_shape=jax.ShapeDtypeStruct((V, D), _f32),
        )(h, agg, W_msg[l, :D], W_msg[l, D:], b_msg[l].reshape(1, D))

    P = pl.pallas_call(
        _head_body,
        out_shape=jax.ShapeDtypeStruct((V, 1), _f32),
    )(h, x_inp_core, W1[:D], W1[D:], b1.reshape(1, D), W2, b2.reshape(1, 1),
      n2g2)
    return P.reshape(V)
